# Initial kernel scaffold; baseline (speedup 1.0000x reference)
import jax
import jax.numpy as jnp
from jax.experimental import pallas as pl


def kernel(hidden_states, first_token_indices, last_token_indices, prompt_lens):
    raise NotImplementedError("write your pallas kernel here")


# speedup vs baseline: 3.5539x; 3.5539x over previous
import functools

import jax
import jax.numpy as jnp
from jax import lax
from jax.experimental import pallas as pl
from jax.experimental.pallas import tpu as pltpu
from jax.experimental.pallas import tpu_sc as plsc

TOK = 32768
D = 1024
B = 16
NC = 2
NS = 16
NW = NC * NS
NL = 16
ROWS_PER_W = TOK // NW
CH = 32
NCHUNK = ROWS_PER_W // CH
DJ = D // NL


def _sc_body(hs_hbm, first_hbm, out_hbm, first_v, acc, buf, sem0, sem1):
    wid = lax.axis_index("s") * NC + lax.axis_index("c")
    base = wid * ROWS_PER_W

    pltpu.sync_copy(first_hbm, first_v.at[pl.ds(0, B)])

    def first_at(i):
        return first_v[pl.ds(i, NL)][0]

    def seg0_body(b, s):
        return s + jnp.where(first_at(b) <= base, 1, 0)

    seg_init = lax.fori_loop(0, B, seg0_body, 0) - 1

    zeros = jnp.zeros((NL,), jnp.float32)

    def zero_body(rb, carry):
        for j in range(DJ):
            acc[rb, pl.ds(j * NL, NL)] = zeros
        return carry

    lax.fori_loop(0, B, zero_body, 0)

    pltpu.async_copy(hs_hbm.at[pl.ds(base, CH)], buf.at[0], sem0)
    pltpu.async_copy(hs_hbm.at[pl.ds(base + CH, CH)], buf.at[1], sem1)
    sems = (sem0, sem1)

    def pair_body(cp, seg):
        ci0 = cp * 2
        for p in range(2):
            ci = ci0 + p
            c0 = base + ci * CH
            pltpu.make_async_copy(
                hs_hbm.at[pl.ds(c0, CH)], buf.at[p], sems[p]).wait()

            def row_body(r, seg):
                tok = c0 + r
                nxt = first_at(jnp.minimum(seg + 1, B - 1))
                seg = jnp.where((tok == nxt) & (seg < B - 1), seg + 1, seg)
                for j in range(DJ):
                    sl = pl.ds(j * NL, NL)
                    plsc.addupdate(acc.at[seg, sl], buf[p, r, sl])
                return seg

            seg = lax.fori_loop(0, CH, row_body, seg)

            @pl.when(ci + 2 < NCHUNK)
            def _():
                pltpu.async_copy(
                    hs_hbm.at[pl.ds(c0 + 2 * CH, CH)], buf.at[p], sems[p])
        return seg

    lax.fori_loop(0, NCHUNK // 2, pair_body, seg_init)

    pltpu.sync_copy(acc, out_hbm.at[wid])


@jax.jit
def _segment_partials(hidden_states, first_token_indices):
    mesh = plsc.VectorSubcoreMesh(
        core_axis_name="c", subcore_axis_name="s",
        num_cores=NC, num_subcores=NS)
    return pl.kernel(
        _sc_body,
        out_type=jax.ShapeDtypeStruct((NW, B, D), jnp.float32),
        mesh=mesh,
        scratch_types=[
            pltpu.VMEM((2 * B,), jnp.int32),
            pltpu.VMEM((B, D), jnp.float32),
            pltpu.VMEM((2, CH, D), jnp.float32),
            pltpu.SemaphoreType.DMA,
            pltpu.SemaphoreType.DMA,
        ],
    )(hidden_states, first_token_indices)


def kernel(hidden_states, first_token_indices, last_token_indices,
           prompt_lens):
    partials = _segment_partials(hidden_states, first_token_indices)
    summed = jnp.sum(partials, axis=0)
    return summed / prompt_lens[:, None].astype(jnp.float32)


# speedup vs baseline: 10.3690x; 2.9176x over previous
import functools

import jax
import jax.numpy as jnp
from jax import lax
from jax.experimental import pallas as pl
from jax.experimental.pallas import tpu as pltpu
from jax.experimental.pallas import tpu_sc as plsc

TOK = 32768
D = 1024
B = 16
NC = 2
NS = 16
NW = NC * NS
NL = 16
ROWS_PER_W = TOK // NW
CH = 32
NCHUNK = ROWS_PER_W // CH
DJ = D // NL
HG = DJ // 2


def _sc_body(hs_hbm, first_hbm, out_hbm, first_v, acc, buf, sem0, sem1):
    wid = lax.axis_index("s") * NC + lax.axis_index("c")
    base = wid * ROWS_PER_W

    pltpu.sync_copy(first_hbm, first_v.at[pl.ds(0, B)])

    def first_at(i):
        return first_v[pl.ds(i, NL)][0]

    def seg0_body(b, s):
        return s + jnp.where(first_at(b) <= base, 1, 0)

    seg_init = lax.fori_loop(0, B, seg0_body, 0) - 1

    zeros = jnp.zeros((NL,), jnp.float32)

    def zero_body(rb, carry):
        for j in range(DJ):
            acc[rb, pl.ds(j * NL, NL)] = zeros
        return carry

    lax.fori_loop(0, B, zero_body, 0)

    pltpu.async_copy(hs_hbm.at[pl.ds(base, CH)], buf.at[0], sem0)
    pltpu.async_copy(hs_hbm.at[pl.ds(base + CH, CH)], buf.at[1], sem1)
    sems = (sem0, sem1)

    def pair_body(cp, seg):
        ci0 = cp * 2
        for p in range(2):
            ci = ci0 + p
            c0 = base + ci * CH
            pltpu.make_async_copy(
                hs_hbm.at[pl.ds(c0, CH)], buf.at[p], sems[p]).wait()

            nxt = jnp.where(seg < B - 1,
                            first_at(jnp.minimum(seg + 1, B - 1)), TOK)

            def fast_path(seg):
                for h in range(2):
                    def rb(r, accs):
                        return tuple(
                            accs[j] + buf[p, r, pl.ds((h * HG + j) * NL, NL)]
                            for j in range(HG))
                    accs = lax.fori_loop(
                        0, CH, rb,
                        tuple(zeros for _ in range(HG)))
                    for j in range(HG):
                        plsc.addupdate(
                            acc.at[seg, pl.ds((h * HG + j) * NL, NL)],
                            accs[j])
                return seg

            def slow_path(seg):
                def row_body(r, seg):
                    tok = c0 + r
                    nx = first_at(jnp.minimum(seg + 1, B - 1))
                    seg = jnp.where((tok == nx) & (seg < B - 1),
                                    seg + 1, seg)
                    for j in range(DJ):
                        sl = pl.ds(j * NL, NL)
                        plsc.addupdate(acc.at[seg, sl], buf[p, r, sl])
                    return seg

                return lax.fori_loop(0, CH, row_body, seg)

            seg = lax.cond(c0 + CH <= nxt, fast_path, slow_path, seg)

            @pl.when(ci + 2 < NCHUNK)
            def _():
                pltpu.async_copy(
                    hs_hbm.at[pl.ds(c0 + 2 * CH, CH)], buf.at[p], sems[p])
        return seg

    lax.fori_loop(0, NCHUNK // 2, pair_body, seg_init)

    pltpu.sync_copy(acc, out_hbm.at[wid])


@jax.jit
def _segment_partials(hidden_states, first_token_indices):
    mesh = plsc.VectorSubcoreMesh(
        core_axis_name="c", subcore_axis_name="s",
        num_cores=NC, num_subcores=NS)
    return pl.kernel(
        _sc_body,
        out_type=jax.ShapeDtypeStruct((NW, B, D), jnp.float32),
        mesh=mesh,
        scratch_types=[
            pltpu.VMEM((2 * B,), jnp.int32),
            pltpu.VMEM((B, D), jnp.float32),
            pltpu.VMEM((2, CH, D), jnp.float32),
            pltpu.SemaphoreType.DMA,
            pltpu.SemaphoreType.DMA,
        ],
    )(hidden_states, first_token_indices)


def kernel(hidden_states, first_token_indices, last_token_indices,
           prompt_lens):
    partials = _segment_partials(hidden_states, first_token_indices)
    summed = jnp.sum(partials, axis=0)
    return summed / prompt_lens[:, None].astype(jnp.float32)


# speedup vs baseline: 13.3699x; 1.2894x over previous
import jax
import jax.numpy as jnp
from jax import lax
from jax.experimental import pallas as pl
from jax.experimental.pallas import tpu as pltpu
from jax.experimental.pallas import tpu_sc as plsc

TOK = 32768
D = 1024
B = 16
NC = 2
NS = 16
NW = NC * NS
NL = 16
SC_ROWS = 16384
ROWS_PER_W = SC_ROWS // NW
CH = 32
NCHUNK = max(ROWS_PER_W // CH, 1)
DJ = D // NL
HG = DJ // 2
TCT = 512


def _sc_body(hs_hbm, first_hbm, out_hbm, first_v, acc, buf, sem0, sem1):
    wid = lax.axis_index("s") * NC + lax.axis_index("c")
    base = wid * ROWS_PER_W

    pltpu.sync_copy(first_hbm, first_v.at[pl.ds(0, B)])

    def first_at(i):
        return first_v[pl.ds(i, NL)][0]

    def seg0_body(b, s):
        return s + jnp.where(first_at(b) <= base, 1, 0)

    seg_init = lax.fori_loop(0, B, seg0_body, 0) - 1

    zeros = jnp.zeros((NL,), jnp.float32)

    def zero_body(rb, carry):
        for j in range(DJ):
            acc[rb, pl.ds(j * NL, NL)] = zeros
        return carry

    lax.fori_loop(0, B, zero_body, 0)

    pltpu.async_copy(hs_hbm.at[pl.ds(base, CH)], buf.at[0], sem0)
    pltpu.async_copy(hs_hbm.at[pl.ds(base + CH, CH)], buf.at[1], sem1)
    sems = (sem0, sem1)

    def pair_body(cp, seg):
        ci0 = cp * 2
        for p in range(2):
            ci = ci0 + p
            c0 = base + ci * CH
            pltpu.make_async_copy(
                hs_hbm.at[pl.ds(c0, CH)], buf.at[p], sems[p]).wait()

            nxt = jnp.where(seg < B - 1,
                            first_at(jnp.minimum(seg + 1, B - 1)), TOK)

            def fast_path(seg):
                for h in range(2):
                    def rb(r, accs):
                        return tuple(
                            accs[j] + buf[p, r, pl.ds((h * HG + j) * NL, NL)]
                            for j in range(HG))
                    accs = lax.fori_loop(
                        0, CH, rb,
                        tuple(zeros for _ in range(HG)))
                    for j in range(HG):
                        plsc.addupdate(
                            acc.at[seg, pl.ds((h * HG + j) * NL, NL)],
                            accs[j])
                return seg

            def slow_path(seg):
                def row_body(r, seg):
                    tok = c0 + r
                    nx = first_at(jnp.minimum(seg + 1, B - 1))
                    seg = jnp.where((tok == nx) & (seg < B - 1),
                                    seg + 1, seg)
                    for j in range(DJ):
                        sl = pl.ds(j * NL, NL)
                        plsc.addupdate(acc.at[seg, sl], buf[p, r, sl])
                    return seg

                return lax.fori_loop(0, CH, row_body, seg)

            seg = lax.cond(c0 + CH <= nxt, fast_path, slow_path, seg)

            @pl.when(ci + 2 < NCHUNK)
            def _():
                pltpu.async_copy(
                    hs_hbm.at[pl.ds(c0 + 2 * CH, CH)], buf.at[p], sems[p])
        return seg

    lax.fori_loop(0, NCHUNK // 2, pair_body, seg_init)

    pltpu.sync_copy(acc, out_hbm.at[wid])


@jax.jit
def _segment_partials(hidden_states, first_token_indices):
    mesh = plsc.VectorSubcoreMesh(
        core_axis_name="c", subcore_axis_name="s",
        num_cores=NC, num_subcores=NS)
    return pl.kernel(
        _sc_body,
        out_type=jax.ShapeDtypeStruct((NW, B, D), jnp.float32),
        mesh=mesh,
        scratch_types=[
            pltpu.VMEM((2 * B,), jnp.int32),
            pltpu.VMEM((B, D), jnp.float32),
            pltpu.VMEM((2, CH, D), jnp.float32),
            pltpu.SemaphoreType.DMA,
            pltpu.SemaphoreType.DMA,
        ],
    )(hidden_states, first_token_indices)


def _tc_body(first_ref, last_ref, x_ref, o_ref):
    i = pl.program_id(0)
    base_tok = SC_ROWS + i * TCT
    tok = base_tok + lax.broadcasted_iota(jnp.int32, (B, TCT), 1)
    mask = ((tok >= first_ref[...]) & (tok <= last_ref[...]))

    @pl.when(i == 0)
    def _():
        o_ref[...] = jnp.zeros_like(o_ref)

    o_ref[...] += jnp.dot(mask.astype(jnp.float32), x_ref[...],
                          preferred_element_type=jnp.float32)


@jax.jit
def _tc_partial(hidden_states, first_token_indices, last_token_indices):
    nt = (TOK - SC_ROWS) // TCT
    return pl.pallas_call(
        _tc_body,
        grid=(nt,),
        in_specs=[
            pl.BlockSpec((B, 1), lambda i: (0, 0)),
            pl.BlockSpec((B, 1), lambda i: (0, 0)),
            pl.BlockSpec((TCT, D), lambda i: (i + SC_ROWS // TCT, 0)),
        ],
        out_specs=pl.BlockSpec((B, D), lambda i: (0, 0)),
        out_shape=jax.ShapeDtypeStruct((B, D), jnp.float32),
    )(first_token_indices[:, None], last_token_indices[:, None],
      hidden_states)


def kernel(hidden_states, first_token_indices, last_token_indices,
           prompt_lens):
    partials = _segment_partials(hidden_states, first_token_indices)
    summed = _tc_partial(hidden_states, first_token_indices,
                         last_token_indices)
    summed = summed + jnp.sum(partials, axis=0)
    return summed / prompt_lens[:, None].astype(jnp.float32)
